# speedup vs baseline: 1.4383x; 1.4383x over previous
import functools

import jax
import jax.numpy as jnp
from jax import lax
from jax.experimental import pallas as pl
from jax.experimental.pallas import tpu as pltpu
from jax.experimental.pallas import tpu_sc as plsc

B = 10000
K = 32
D = 128
ED = 128
NC, NS = 2, 16
NW = NC * NS
BPW = 320
BP = NW * BPW
CHUNK = 128
RPC = CHUNK // K
NCH = (BPW * K) // CHUNK
LANES = 16
DV = D // LANES

_mesh = plsc.VectorSubcoreMesh(core_axis_name="c", subcore_axis_name="s")


@functools.partial(
    pl.kernel,
    out_type=jax.ShapeDtypeStruct((BP, D), jnp.float32),
    mesh=_mesh,
    scratch_types=[
        pltpu.VMEM((NCH, CHUNK), jnp.int32),
        pltpu.VMEM((2, CHUNK, D), jnp.float32),
        pltpu.VMEM((BPW, D), jnp.float32),
        pltpu.SemaphoreType.DMA,
        pltpu.SemaphoreType.DMA,
    ],
)
def _sc_gather_mean(nbr_hbm, table_hbm, agg_hbm, idx_v, rows_v, obuf, sem0, sem1):
    wid = lax.axis_index("s") * NC + lax.axis_index("c")
    sems = (sem0, sem1)

    pltpu.sync_copy(nbr_hbm.at[wid], idx_v)

    def gather_start(c, slot):
        pltpu.async_copy(table_hbm.at[idx_v.at[c]], rows_v.at[slot], sems[slot])

    def gather_wait(slot):
        pltpu.make_async_copy(
            table_hbm.at[idx_v.at[0]], rows_v.at[slot], sems[slot]
        ).wait()

    def accum(c, slot):
        for r in range(RPC):
            def body(k, acc):
                row = r * K + k
                return tuple(
                    acc[d] + rows_v[slot, row, pl.ds(d * LANES, LANES)]
                    for d in range(DV)
                )
            acc = lax.fori_loop(
                0, K, body,
                tuple(jnp.zeros((LANES,), jnp.float32) for _ in range(DV)),
            )
            orow = c * RPC + r
            for d in range(DV):
                obuf[orow, pl.ds(d * LANES, LANES)] = acc[d] * (1.0 / K)

    gather_start(0, 0)
    gather_start(1, 1)

    def outer(c0, carry):
        for slot in range(2):
            c = c0 * 2 + slot
            gather_wait(slot)
            accum(c, slot)

            @pl.when(c + 2 < NCH)
            def _():
                gather_start(c + 2, slot)
        return carry

    lax.fori_loop(0, NCH // 2, outer, 0)

    pltpu.sync_copy(obuf, agg_hbm.at[pl.ds(wid * BPW, BPW)])


def _tc_body(w_ref, a_ref, o_ref):
    o_ref[...] = jnp.maximum(
        lax.dot_general(
            w_ref[...], a_ref[...],
            dimension_numbers=(((1,), (1,)), ((), ())),
            preferred_element_type=jnp.float32,
        ),
        0.0,
    )


_BN = 1024

_tc_matmul = pl.pallas_call(
    _tc_body,
    grid=(BP // _BN,),
    in_specs=[
        pl.BlockSpec((ED, D), lambda i: (0, 0)),
        pl.BlockSpec((_BN, D), lambda i: (i, 0)),
    ],
    out_specs=pl.BlockSpec((ED, _BN), lambda i: (0, i)),
    out_shape=jax.ShapeDtypeStruct((ED, BP), jnp.float32),
)


def kernel(nodes, all_neighbors, feat_table, weight):
    del nodes
    nbr = all_neighbors.astype(jnp.int32)
    nbr = jnp.pad(nbr, ((0, BP - B), (0, 0))).reshape(NW, NCH, CHUNK)
    agg = _sc_gather_mean(nbr, feat_table)
    out = _tc_matmul(weight, agg)
    return out[:, :B]


# speedup vs baseline: 1.4524x; 1.0098x over previous
import functools

import jax
import jax.numpy as jnp
from jax import lax
from jax.experimental import pallas as pl
from jax.experimental.pallas import tpu as pltpu
from jax.experimental.pallas import tpu_sc as plsc

B = 10000
K = 32
D = 128
ED = 128
NC, NS = 2, 16
NW = NC * NS
BPW = 320
BP = NW * BPW
CHUNK = 128
RPC = CHUNK // K
NCH = (BPW * K) // CHUNK
LANES = 16
DV = D // LANES
NBUF = 4

_mesh = plsc.VectorSubcoreMesh(core_axis_name="c", subcore_axis_name="s")


@functools.partial(
    pl.kernel,
    out_type=jax.ShapeDtypeStruct((BP, D), jnp.float32),
    mesh=_mesh,
    scratch_types=[
        pltpu.VMEM((NCH, CHUNK), jnp.int32),
        pltpu.VMEM((NBUF, CHUNK, D), jnp.float32),
        pltpu.VMEM((BPW, D), jnp.float32),
        pltpu.SemaphoreType.DMA,
        pltpu.SemaphoreType.DMA,
        pltpu.SemaphoreType.DMA,
        pltpu.SemaphoreType.DMA,
    ],
)
def _sc_gather_mean(nbr_hbm, table_hbm, agg_hbm, idx_v, rows_v, obuf,
                    sem0, sem1, sem2, sem3):
    wid = lax.axis_index("s") * NC + lax.axis_index("c")
    sems = (sem0, sem1, sem2, sem3)

    pltpu.sync_copy(nbr_hbm.at[wid], idx_v)

    def gather_start(c, slot):
        pltpu.async_copy(table_hbm.at[idx_v.at[c]], rows_v.at[slot], sems[slot])

    def gather_wait(slot):
        pltpu.make_async_copy(
            table_hbm.at[idx_v.at[0]], rows_v.at[slot], sems[slot]
        ).wait()

    def accum(c, slot):
        for r in range(RPC):
            def body(k, acc):
                row = r * K + k
                return tuple(
                    acc[d] + rows_v[slot, row, pl.ds(d * LANES, LANES)]
                    for d in range(DV)
                )
            acc = lax.fori_loop(
                0, K, body,
                tuple(jnp.zeros((LANES,), jnp.float32) for _ in range(DV)),
            )
            orow = c * RPC + r
            for d in range(DV):
                obuf[orow, pl.ds(d * LANES, LANES)] = acc[d] * (1.0 / K)

    for p in range(NBUF - 1):
        gather_start(p, p)

    def outer(c0, carry):
        for b in range(NBUF):
            c = c0 * NBUF + b
            gather_wait(b)

            @pl.when(c + NBUF - 1 < NCH)
            def _():
                gather_start(c + NBUF - 1, (b + NBUF - 1) % NBUF)

            accum(c, b)
        return carry

    lax.fori_loop(0, NCH // NBUF, outer, 0)

    pltpu.sync_copy(obuf, agg_hbm.at[pl.ds(wid * BPW, BPW)])


def _tc_body(w_ref, a_ref, o_ref):
    o_ref[...] = jnp.maximum(
        lax.dot_general(
            w_ref[...], a_ref[...],
            dimension_numbers=(((1,), (1,)), ((), ())),
            preferred_element_type=jnp.float32,
        ),
        0.0,
    )


_BN = 1024

_tc_matmul = pl.pallas_call(
    _tc_body,
    grid=(BP // _BN,),
    in_specs=[
        pl.BlockSpec((ED, D), lambda i: (0, 0)),
        pl.BlockSpec((_BN, D), lambda i: (i, 0)),
    ],
    out_specs=pl.BlockSpec((ED, _BN), lambda i: (0, i)),
    out_shape=jax.ShapeDtypeStruct((ED, BP), jnp.float32),
)


def kernel(nodes, all_neighbors, feat_table, weight):
    del nodes
    nbr = all_neighbors.astype(jnp.int32)
    nbr = jnp.pad(nbr, ((0, BP - B), (0, 0))).reshape(NW, NCH, CHUNK)
    agg = _sc_gather_mean(nbr, feat_table)
    out = _tc_matmul(weight, agg)
    return out[:, :B]


# speedup vs baseline: 1.5404x; 1.0606x over previous
import functools

import jax
import jax.numpy as jnp
from jax import lax
from jax.experimental import pallas as pl
from jax.experimental.pallas import tpu as pltpu
from jax.experimental.pallas import tpu_sc as plsc

B = 10000
K = 32
D = 128
ED = 128
NC, NS = 2, 16
BP = 10240
CHUNK = 128
RPC = CHUNK // K
TOTAL_CH = BP * K // CHUNK
C0 = 128
C1 = 32
LANES = 16
DV = D // LANES
NBUF = 2

_mesh = plsc.VectorSubcoreMesh(core_axis_name="c", subcore_axis_name="s")


@functools.partial(
    pl.kernel,
    out_type=jax.ShapeDtypeStruct((BP, D), jnp.float32),
    mesh=_mesh,
    scratch_types=[
        pltpu.VMEM((C0, CHUNK), jnp.int32),
        pltpu.VMEM((NBUF, CHUNK, D), jnp.float32),
        pltpu.VMEM((C0 * RPC, D), jnp.float32),
        pltpu.SemaphoreType.DMA,
        pltpu.SemaphoreType.DMA,
    ],
)
def _sc_gather_mean(nbr_hbm, table_hbm, agg_hbm, idx_v, rows_v, obuf, sem0, sem1):
    cid = lax.axis_index("c")
    sid = lax.axis_index("s")
    sems = (sem0, sem1)

    def gather_start(c, slot):
        pltpu.async_copy(table_hbm.at[idx_v.at[c]], rows_v.at[slot], sems[slot])

    def gather_wait(slot):
        pltpu.make_async_copy(
            table_hbm.at[idx_v.at[0]], rows_v.at[slot], sems[slot]
        ).wait()

    def accum(c, slot):
        for r in range(RPC):
            def body(k, acc):
                row = r * K + k
                return tuple(
                    acc[d] + rows_v[slot, row, pl.ds(d * LANES, LANES)]
                    for d in range(DV)
                )
            acc = lax.fori_loop(
                0, K, body,
                tuple(jnp.zeros((LANES,), jnp.float32) for _ in range(DV)),
            )
            orow = c * RPC + r
            for d in range(DV):
                obuf[orow, pl.ds(d * LANES, LANES)] = acc[d] * (1.0 / K)

    def run(nch, cbase):
        pltpu.sync_copy(
            nbr_hbm.at[pl.ds(cbase, nch)], idx_v.at[pl.ds(0, nch)]
        )
        for p in range(NBUF - 1):
            gather_start(p, p)

        def outer(c0, carry):
            for b in range(NBUF):
                c = c0 * NBUF + b
                gather_wait(b)

                @pl.when(c + NBUF - 1 < nch)
                def _():
                    gather_start(c + NBUF - 1, (b + NBUF - 1) % NBUF)

                accum(c, b)
            return carry

        lax.fori_loop(0, nch // NBUF, outer, 0)
        pltpu.sync_copy(
            obuf.at[pl.ds(0, nch * RPC)],
            agg_hbm.at[pl.ds(cbase * RPC, nch * RPC)],
        )

    @pl.when(cid == 0)
    def _():
        run(C0, sid * C0)

    @pl.when(cid == 1)
    def _():
        run(C1, NS * C0 + sid * C1)


def _tc_body(w_ref, a_ref, o_ref):
    o_ref[...] = jnp.maximum(
        lax.dot_general(
            w_ref[...], a_ref[...],
            dimension_numbers=(((1,), (1,)), ((), ())),
            preferred_element_type=jnp.float32,
        ),
        0.0,
    )


_BN = 1024

_tc_matmul = pl.pallas_call(
    _tc_body,
    grid=(BP // _BN,),
    in_specs=[
        pl.BlockSpec((ED, D), lambda i: (0, 0)),
        pl.BlockSpec((_BN, D), lambda i: (i, 0)),
    ],
    out_specs=pl.BlockSpec((ED, _BN), lambda i: (0, i)),
    out_shape=jax.ShapeDtypeStruct((ED, BP), jnp.float32),
)


def kernel(nodes, all_neighbors, feat_table, weight):
    del nodes
    nbr = all_neighbors.astype(jnp.int32)
    nbr = jnp.pad(nbr, ((0, BP - B), (0, 0))).reshape(TOTAL_CH, CHUNK)
    agg = _sc_gather_mean(nbr, feat_table)
    out = _tc_matmul(weight, agg)
    return out[:, :B]
